# baseline (speedup 1.0000x reference)
import jax
import jax.numpy as jnp
from jax import lax
from jax.experimental import pallas as pl
from jax.experimental.pallas import tpu as pltpu
from jax.experimental.pallas import tpu_sc as plsc

_B = 16384
_H = 50
_D = 32
_N = _B * _H

_NC, _NS = 2, 16
_NW = _NC * _NS
_BPW = _B // _NW
_CH = _BPW * _D
_HSTRIDE = 4 * 131072


def _permute_block(rows_v, perm_v, iota16):
    def body(m0, carry):
        col = (iota16 + m0) & 31
        dstv = ((col >> 3) << 12) + ((col & 7) << 7) + iota16
        for btp in range(4):
            for blg in range(8):
                row = iota16 + (btp * 128 + blg * 16)
                v = plsc.load_gather(rows_v, [row, col])
                plsc.store_scatter(perm_v, [dstv + (btp * 1024 + blg * 16)], v)
        return carry

    lax.fori_loop(0, _D, body, 0)


def _gather_body(idx_hbm, table_hbm, out_hbm,
                 idx0, idx1, rows0, rows1, perm0, perm1,
                 isem0, isem1, gsem0, gsem1, wsem0, wsem1):
    wid = lax.axis_index("s") * _NC + lax.axis_index("c")
    b0 = wid * _BPW
    out_base = wid * 4096
    iota16 = lax.iota(jnp.int32, 16)

    def icp(h, idx_v, sem):
        off = pl.multiple_of(h * _B + b0, 8)
        return pltpu.make_async_copy(idx_hbm.at[pl.ds(off, _BPW)], idx_v, sem)

    def gcp(idx_v, rows_v, sem):
        return pltpu.make_async_copy(table_hbm.at[idx_v], rows_v, sem)

    def wcp(h, dt, perm_v, sem):
        off = pl.multiple_of(h * _HSTRIDE + dt * 131072 + out_base, 8)
        return pltpu.make_async_copy(
            perm_v.at[pl.ds(dt * 4096, 4096)],
            out_hbm.at[pl.ds(off, 4096)],
            sem)

    def chunk(c, idx_c2, idx_o, rows_c, rows_o, perm_c,
              isem_c2, isem_o, gsem_c, gsem_o, wsem_c, first, last):
        @pl.when(c + 1 < _H)
        def _next():
            icp(0, idx_o, isem_o).wait()
            gcp(idx_o, rows_o, gsem_o).start()

        gcp(idx_c2, rows_c, gsem_c).wait()

        @pl.when(c + 2 < _H)
        def _stage():
            icp(c + 2, idx_c2, isem_c2).start()

        @pl.when(jnp.logical_not(first))
        def _drain():
            for dt in range(4):
                wcp(0, dt, perm_c, wsem_c).wait()

        _permute_block(rows_c, perm_c, iota16)
        for dt in range(4):
            wcp(c, dt, perm_c, wsem_c).start()

    icp(0, idx0, isem0).start()
    icp(1, idx1, isem1).start()
    icp(0, idx0, isem0).wait()
    gcp(idx0, rows0, gsem0).start()

    def pair(j, carry):
        c0 = 2 * j
        chunk(c0, idx0, idx1, rows0, rows1, perm0,
              isem0, isem1, gsem0, gsem1, wsem0, j == 0, False)
        chunk(c0 + 1, idx1, idx0, rows1, rows0, perm1,
              isem1, isem0, gsem1, gsem0, wsem1, j == 0, False)
        return carry

    lax.fori_loop(0, _H // 2, pair, 0)
    for dt in range(4):
        wcp(0, dt, perm0, wsem0).wait()
        wcp(0, dt, perm1, wsem1).wait()


_CW = 256
_NCHUNKS = 999936 // _CW
_CBASE = _NCHUNKS // _NW
_CEXTRA = _NCHUNKS - _CBASE * _NW


def _relayout_transpose(stg, perm, iota16):
    def body(m0, carry):
        rowv = (iota16 + m0) & 31
        dstb = iota16 * 32 + rowv
        for rc in range(_CW // 16):
            colv = iota16 + rc * 16
            v = plsc.load_gather(stg, [rowv, colv])
            plsc.store_scatter(perm, [dstb + rc * 512], v)
        return carry

    lax.fori_loop(0, 32, body, 0)


def _relayout_body(tableT_hbm, out_hbm,
                   stg0, stg1, stg_t, perm0, perm1,
                   isem0, isem1, wsem0, wsem1):
    wid = lax.axis_index("s") * _NC + lax.axis_index("c")
    start = wid * _CBASE + jnp.minimum(wid, _CEXTRA)
    count = _CBASE + (wid < _CEXTRA).astype(jnp.int32)
    iota16 = lax.iota(jnp.int32, 16)

    def c0_of(g):
        return pl.multiple_of(g * _CW, _CW)

    def icp(g, stg, sem):
        return pltpu.make_async_copy(
            tableT_hbm.at[:, pl.ds(c0_of(g), _CW)], stg, sem)

    def wcp(g, perm, sem):
        off = pl.multiple_of(c0_of(g) * _D, 64)
        return pltpu.make_async_copy(perm,
                                     out_hbm.at[pl.ds(off, _CW * _D)], sem)

    def do_chunk(i, stg_m, perm_m, isem_m, wsem_m, stg_o, isem_o):
        g = start + i

        @pl.when(i + 1 < count)
        def _stage():
            icp(g + 1, stg_o, isem_o).start()

        icp(g, stg_m, isem_m).wait()

        @pl.when(i >= 2)
        def _drain():
            wcp(g, perm_m, wsem_m).wait()

        _relayout_transpose(stg_m, perm_m, iota16)
        wcp(g, perm_m, wsem_m).start()

    icp(start, stg0, isem0).start()

    def step(i, carry):
        @pl.when((i & 1) == 0)
        def _even():
            do_chunk(i, stg0, perm0, isem0, wsem0, stg1, isem1)

        @pl.when((i & 1) == 1)
        def _odd():
            do_chunk(i, stg1, perm1, isem1, wsem1, stg0, isem0)

        return carry

    lax.fori_loop(0, count, step, 0)
    wcp(0, perm0, wsem0).wait()
    wcp(0, perm1, wsem1).wait()

    @pl.when(wid == _NW - 1)
    def _tail():
        pltpu.sync_copy(tableT_hbm.at[:, pl.ds(999936, 64)], stg_t)

        def tbody(m0, carry):
            rowv = (iota16 + m0) & 31
            dstb = iota16 * 32 + rowv
            for rc in range(4):
                colv = iota16 + rc * 16
                v = plsc.load_gather(stg_t, [rowv, colv])
                plsc.store_scatter(perm0, [dstb + rc * 512], v)
            return carry

        lax.fori_loop(0, 32, tbody, 0)
        pltpu.sync_copy(perm0.at[pl.ds(0, 2048)],
                        out_hbm.at[pl.ds(999936 * _D, 2048)])


@jax.jit
def _table_relayout(tableT):
    mesh = plsc.VectorSubcoreMesh(core_axis_name="c", subcore_axis_name="s")
    f = pl.kernel(
        _relayout_body,
        out_type=jax.ShapeDtypeStruct((1000000 * _D,), jnp.float32),
        scratch_types=[
            pltpu.VMEM((_D, _CW), jnp.float32),
            pltpu.VMEM((_D, _CW), jnp.float32),
            pltpu.VMEM((_D, 64), jnp.float32),
            pltpu.VMEM((_CW * _D,), jnp.float32),
            pltpu.VMEM((_CW * _D,), jnp.float32),
            pltpu.SemaphoreType.DMA,
            pltpu.SemaphoreType.DMA,
            pltpu.SemaphoreType.DMA,
            pltpu.SemaphoreType.DMA,
        ],
        mesh=mesh,
        compiler_params=pltpu.CompilerParams(
            use_tc_tiling_on_sc=True, needs_layout_passes=False),
    )
    return f(tableT)


@jax.jit
def _embedding_gather(idx, table):
    mesh = plsc.VectorSubcoreMesh(core_axis_name="c", subcore_axis_name="s")
    f = pl.kernel(
        _gather_body,
        out_type=jax.ShapeDtypeStruct((_N * _D,), jnp.float32),
        scratch_types=[
            pltpu.VMEM((_BPW,), jnp.int32),
            pltpu.VMEM((_BPW,), jnp.int32),
            pltpu.VMEM((_BPW, _D), jnp.float32),
            pltpu.VMEM((_BPW, _D), jnp.float32),
            pltpu.VMEM((_CH,), jnp.float32),
            pltpu.VMEM((_CH,), jnp.float32),
            pltpu.SemaphoreType.DMA,
            pltpu.SemaphoreType.DMA,
            pltpu.SemaphoreType.DMA,
            pltpu.SemaphoreType.DMA,
            pltpu.SemaphoreType.DMA,
            pltpu.SemaphoreType.DMA,
        ],
        mesh=mesh,
        compiler_params=pltpu.CompilerParams(
            use_tc_tiling_on_sc=False, needs_layout_passes=False),
    )
    return f(idx, table)


def kernel(text, table):
    idx = text.T.reshape(-1).astype(jnp.int32)
    tlin = _table_relayout(table.T)
    outflat = _embedding_gather(idx, tlin.reshape(1000000, _D))
    out5 = outflat.reshape(_H, 4, 128, 8, 128)
    return out5.transpose(2, 4, 0, 1, 3).reshape(_B, _H, _D)


# speedup vs baseline: 1.0081x; 1.0081x over previous
import jax
import jax.numpy as jnp
from jax import lax
from jax.experimental import pallas as pl
from jax.experimental.pallas import tpu as pltpu
from jax.experimental.pallas import tpu_sc as plsc

_B = 16384
_H = 50
_D = 32
_N = _B * _H

_NC, _NS = 2, 16
_NW = _NC * _NS
_BPW = _B // _NW
_CH = _BPW * _D
_HSTRIDE = 4 * 131072


def _permute_block(rows_v, perm_v, iota16):
    def body(m0, carry):
        col = (iota16 + m0) & 31
        dstv = ((col >> 3) << 12) + ((col & 7) << 7) + iota16
        for btp in range(4):
            for blg in range(8):
                row = iota16 + (btp * 128 + blg * 16)
                v = plsc.load_gather(rows_v, [row, col])
                plsc.store_scatter(perm_v, [dstv + (btp * 1024 + blg * 16)], v)
        return carry

    lax.fori_loop(0, _D, body, 0)


def _gather_body(idx_hbm, table_hbm, out_hbm,
                 idx0, idx1, rows0, rows1, perm0, perm1,
                 isem0, isem1, gsem0, gsem1, wsem0, wsem1):
    wid = lax.axis_index("s") * _NC + lax.axis_index("c")
    b0 = wid * _BPW
    out_base = wid * 4096
    iota16 = lax.iota(jnp.int32, 16)

    def icp(h, idx_v, sem):
        off = pl.multiple_of(h * _B + b0, 8)
        return pltpu.make_async_copy(idx_hbm.at[pl.ds(off, _BPW)], idx_v, sem)

    def gcp(idx_v, rows_v, sem):
        return pltpu.make_async_copy(table_hbm.at[idx_v], rows_v, sem)

    def wcp(h, dt, perm_v, sem):
        off = pl.multiple_of(h * _HSTRIDE + dt * 131072 + out_base, 8)
        return pltpu.make_async_copy(
            perm_v.at[pl.ds(dt * 4096, 4096)],
            out_hbm.at[pl.ds(off, 4096)],
            sem)

    def chunk(c, idx_c2, idx_o, rows_c, rows_o, perm_c,
              isem_c2, isem_o, gsem_c, gsem_o, wsem_c, first, last):
        gcp(idx_c2, rows_c, gsem_c).wait()

        @pl.when(c + 2 < _H)
        def _stage():
            icp(c + 2, idx_c2, isem_c2).start()

        @pl.when(c + 1 < _H)
        def _next():
            icp(0, idx_o, isem_o).wait()
            gcp(idx_o, rows_o, gsem_o).start()

        @pl.when(jnp.logical_not(first))
        def _drain():
            for dt in range(4):
                wcp(0, dt, perm_c, wsem_c).wait()

        _permute_block(rows_c, perm_c, iota16)
        for dt in range(4):
            wcp(c, dt, perm_c, wsem_c).start()

    icp(0, idx0, isem0).start()
    icp(1, idx1, isem1).start()
    icp(0, idx0, isem0).wait()
    gcp(idx0, rows0, gsem0).start()

    def pair(j, carry):
        c0 = 2 * j
        chunk(c0, idx0, idx1, rows0, rows1, perm0,
              isem0, isem1, gsem0, gsem1, wsem0, j == 0, False)
        chunk(c0 + 1, idx1, idx0, rows1, rows0, perm1,
              isem1, isem0, gsem1, gsem0, wsem1, j == 0, False)
        return carry

    lax.fori_loop(0, _H // 2, pair, 0)
    for dt in range(4):
        wcp(0, dt, perm0, wsem0).wait()
        wcp(0, dt, perm1, wsem1).wait()


_CW = 256
_NCHUNKS = 999936 // _CW
_CBASE = _NCHUNKS // _NW
_CEXTRA = _NCHUNKS - _CBASE * _NW


def _relayout_transpose(stg, perm, iota16):
    def body(m0, carry):
        rowv = (iota16 + m0) & 31
        dstb = iota16 * 32 + rowv
        for rc in range(_CW // 16):
            colv = iota16 + rc * 16
            v = plsc.load_gather(stg, [rowv, colv])
            plsc.store_scatter(perm, [dstb + rc * 512], v)
        return carry

    lax.fori_loop(0, 32, body, 0)


def _relayout_body(tableT_hbm, out_hbm,
                   stg0, stg1, stg_t, perm0, perm1,
                   isem0, isem1, wsem0, wsem1):
    wid = lax.axis_index("s") * _NC + lax.axis_index("c")
    start = wid * _CBASE + jnp.minimum(wid, _CEXTRA)
    count = _CBASE + (wid < _CEXTRA).astype(jnp.int32)
    iota16 = lax.iota(jnp.int32, 16)

    def c0_of(g):
        return pl.multiple_of(g * _CW, _CW)

    def icp(g, stg, sem):
        return pltpu.make_async_copy(
            tableT_hbm.at[:, pl.ds(c0_of(g), _CW)], stg, sem)

    def wcp(g, perm, sem):
        off = pl.multiple_of(c0_of(g) * _D, 64)
        return pltpu.make_async_copy(perm,
                                     out_hbm.at[pl.ds(off, _CW * _D)], sem)

    def do_chunk(i, stg_m, perm_m, isem_m, wsem_m, stg_o, isem_o):
        g = start + i
        icp(g, stg_m, isem_m).wait()

        @pl.when(i + 1 < count)
        def _stage():
            icp(g + 1, stg_o, isem_o).start()

        @pl.when(i >= 2)
        def _drain():
            wcp(g, perm_m, wsem_m).wait()

        _relayout_transpose(stg_m, perm_m, iota16)
        wcp(g, perm_m, wsem_m).start()

    icp(start, stg0, isem0).start()

    def step(i, carry):
        @pl.when((i & 1) == 0)
        def _even():
            do_chunk(i, stg0, perm0, isem0, wsem0, stg1, isem1)

        @pl.when((i & 1) == 1)
        def _odd():
            do_chunk(i, stg1, perm1, isem1, wsem1, stg0, isem0)

        return carry

    lax.fori_loop(0, count, step, 0)
    wcp(0, perm0, wsem0).wait()
    wcp(0, perm1, wsem1).wait()

    @pl.when(wid == _NW - 1)
    def _tail():
        pltpu.sync_copy(tableT_hbm.at[:, pl.ds(999936, 64)], stg_t)

        def tbody(m0, carry):
            rowv = (iota16 + m0) & 31
            dstb = iota16 * 32 + rowv
            for rc in range(4):
                colv = iota16 + rc * 16
                v = plsc.load_gather(stg_t, [rowv, colv])
                plsc.store_scatter(perm0, [dstb + rc * 512], v)
            return carry

        lax.fori_loop(0, 32, tbody, 0)
        pltpu.sync_copy(perm0.at[pl.ds(0, 2048)],
                        out_hbm.at[pl.ds(999936 * _D, 2048)])


@jax.jit
def _table_relayout(tableT):
    mesh = plsc.VectorSubcoreMesh(core_axis_name="c", subcore_axis_name="s")
    f = pl.kernel(
        _relayout_body,
        out_type=jax.ShapeDtypeStruct((1000000 * _D,), jnp.float32),
        scratch_types=[
            pltpu.VMEM((_D, _CW), jnp.float32),
            pltpu.VMEM((_D, _CW), jnp.float32),
            pltpu.VMEM((_D, 64), jnp.float32),
            pltpu.VMEM((_CW * _D,), jnp.float32),
            pltpu.VMEM((_CW * _D,), jnp.float32),
            pltpu.SemaphoreType.DMA,
            pltpu.SemaphoreType.DMA,
            pltpu.SemaphoreType.DMA,
            pltpu.SemaphoreType.DMA,
        ],
        mesh=mesh,
        compiler_params=pltpu.CompilerParams(
            use_tc_tiling_on_sc=True, needs_layout_passes=False),
    )
    return f(tableT)


@jax.jit
def _embedding_gather(idx, table):
    mesh = plsc.VectorSubcoreMesh(core_axis_name="c", subcore_axis_name="s")
    f = pl.kernel(
        _gather_body,
        out_type=jax.ShapeDtypeStruct((_N * _D,), jnp.float32),
        scratch_types=[
            pltpu.VMEM((_BPW,), jnp.int32),
            pltpu.VMEM((_BPW,), jnp.int32),
            pltpu.VMEM((_BPW, _D), jnp.float32),
            pltpu.VMEM((_BPW, _D), jnp.float32),
            pltpu.VMEM((_CH,), jnp.float32),
            pltpu.VMEM((_CH,), jnp.float32),
            pltpu.SemaphoreType.DMA,
            pltpu.SemaphoreType.DMA,
            pltpu.SemaphoreType.DMA,
            pltpu.SemaphoreType.DMA,
            pltpu.SemaphoreType.DMA,
            pltpu.SemaphoreType.DMA,
        ],
        mesh=mesh,
        compiler_params=pltpu.CompilerParams(
            use_tc_tiling_on_sc=False, needs_layout_passes=False),
    )
    return f(idx, table)


def kernel(text, table):
    idx = text.T.reshape(-1).astype(jnp.int32)
    tlin = _table_relayout(table.T)
    outflat = _embedding_gather(idx, tlin.reshape(1000000, _D))
    out5 = outflat.reshape(_H, 4, 128, 8, 128)
    return out5.transpose(2, 4, 0, 1, 3).reshape(_B, _H, _D)
